# Initial kernel scaffold; baseline (speedup 1.0000x reference)
import jax
import jax.numpy as jnp
from jax.experimental import pallas as pl


def kernel(x):
    raise NotImplementedError("write your pallas kernel here")


# speedup vs baseline: 30.9471x; 30.9471x over previous
import functools

import jax
import jax.numpy as jnp
from jax import lax
from jax.experimental import pallas as pl
from jax.experimental.pallas import tpu as pltpu
from jax.experimental.pallas import tpu_sc as plsc

N = 128 * 4096
NSLICES = 32
BITS = 14
SHIFT = 32 - BITS
NBINS = 1 << BITS
W = 8192
NC, NS = 2, 16


def _sortable_key(f):
    b = lax.bitcast_convert_type(f, jnp.int32)
    m = lax.shift_right_arithmetic(b, 1 + 30)
    return b ^ (m | jnp.int32(-2147483648))


def _sc_ranks(xs):
    mesh = plsc.VectorSubcoreMesh(core_axis_name="c", subcore_axis_name="s")

    @functools.partial(
        pl.kernel,
        mesh=mesh,
        compiler_params=pltpu.CompilerParams(needs_layout_passes=False),
        out_type=jax.ShapeDtypeStruct((NSLICES, N), jnp.int32),
        scratch_types=[
            pltpu.VMEM((NBINS,), jnp.int32),
            pltpu.VMEM((NBINS,), jnp.int32),
            pltpu.VMEM((NBINS,), jnp.float32),
            pltpu.VMEM((W,), jnp.float32),
            pltpu.VMEM((W,), jnp.int32),
        ],
    )
    def ranks_kernel(x_hbm, out_hbm, hist, base, cntf, inbuf, outbuf):
        wid = lax.axis_index("s") * NC + lax.axis_index("c")

        def zero_body(i, _):
            hist[pl.ds(i * 16, 16)] = jnp.zeros((16,), jnp.int32)
            return 0

        lax.fori_loop(0, NBINS // 16, zero_body, 0)

        ones = jnp.ones((16,), jnp.int32)

        def win1(w, _):
            pltpu.sync_copy(x_hbm.at[wid, pl.ds(w * W, W)], inbuf)

            def vec1(v, _):
                u = _sortable_key(inbuf[pl.ds(v * 16, 16)])
                h = lax.shift_right_logical(u, SHIFT)
                plsc.addupdate_scatter(hist, [h], ones)
                return 0

            return lax.fori_loop(0, W // 16, vec1, 0)

        lax.fori_loop(0, N // W, win1, 0)

        def csum(i, carry):
            v = hist[pl.ds(i * 16, 16)]
            s = jnp.cumsum(v)
            base[pl.ds(i * 16, 16)] = (s - v) + carry
            cntf[pl.ds(i * 16, 16)] = v.astype(jnp.float32)
            return carry + jnp.sum(v)

        lax.fori_loop(0, NBINS // 16, csum, jnp.int32(0))

        def win2(w, _):
            pltpu.sync_copy(x_hbm.at[wid, pl.ds(w * W, W)], inbuf)

            def vec2(v, _):
                u = _sortable_key(inbuf[pl.ds(v * 16, 16)])
                h = lax.shift_right_logical(u, SHIFT)
                lo = u & jnp.int32((1 << SHIFT) - 1)
                bs = plsc.load_gather(base, [h])
                cf = plsc.load_gather(cntf, [h])
                t = cf * (lo.astype(jnp.float32) * jnp.float32(1.0 / (1 << SHIFT)))
                t = jnp.minimum(t, cf - jnp.float32(1.0))
                outbuf[pl.ds(v * 16, 16)] = bs + t.astype(jnp.int32)
                return 0

            lax.fori_loop(0, W // 16, vec2, 0)
            pltpu.sync_copy(outbuf, out_hbm.at[wid, pl.ds(w * W, W)])
            return 0

        lax.fori_loop(0, N // W, win2, 0)

    return ranks_kernel(xs)


def _erfinv_poly(w, coeffs):
    p = jnp.float32(coeffs[0])
    for c in coeffs[1:]:
        p = jnp.float32(c) + p * w
    return p


def _gauss_body(r_ref, y_ref):
    r = r_ref[...].astype(jnp.float32)
    q = r * jnp.float32(1.0 / (N - 1))
    q = jnp.clip(q, jnp.float32(1e-05), jnp.float32(1 - 1e-05))
    z = jnp.float32(2.0) * q - jnp.float32(1.0)
    w = -jnp.log(jnp.float32(1.0) - z * z)
    pc = _erfinv_poly(w - jnp.float32(2.5), [
        2.81022636e-08, 3.43273939e-07, -3.5233877e-06, -4.39150654e-06,
        0.00021858087, -0.00125372503, -0.00417768164, 0.246640727,
        1.50140941])
    wt = jnp.sqrt(jnp.maximum(w, jnp.float32(5.0))) - jnp.float32(3.0)
    pt = _erfinv_poly(wt, [
        -0.000200214257, 0.000100950558, 0.00134934322, -0.00367342844,
        0.00573950773, -0.0076224613, 0.00943887047, 1.00167406,
        2.83297682])
    p = jnp.where(w < jnp.float32(5.0), pc, pt)
    y_ref[...] = jnp.float32(2.0 ** 0.5) * (p * z)


def _tc_gauss(r2d):
    rows, cols = r2d.shape
    blk = 256
    return pl.pallas_call(
        _gauss_body,
        out_shape=jax.ShapeDtypeStruct((rows, cols), jnp.float32),
        grid=(rows // blk,),
        in_specs=[pl.BlockSpec((blk, cols), lambda i: (i, 0))],
        out_specs=pl.BlockSpec((blk, cols), lambda i: (i, 0)),
    )(r2d)


def kernel(x):
    xs = x.reshape(NSLICES, N)
    ranks = _sc_ranks(xs)
    y = _tc_gauss(ranks.reshape(4096, 4096))
    return y.reshape(x.shape)
